# baseline (speedup 1.0000x reference)
import functools

import jax
import jax.numpy as jnp
from jax import lax
from jax.experimental import pallas as pl
from jax.experimental.pallas import tpu as pltpu
from jax.experimental.pallas import tpu_sc as plsc

_R = 200
_V = 128
N_COLS = 200


_LN2 = 0.6931471805599453
_SQRT2 = 1.4142135623730951


def _vln(x):
    bits = plsc.bitcast(x, jnp.int32)
    e = (bits >> 23) - 127
    mbits = (bits & 0x007FFFFF) | 0x3F800000
    m = plsc.bitcast(mbits, jnp.float32)
    big = m > _SQRT2
    m = jnp.where(big, m * 0.5, m)
    e = e + jnp.where(big, 1, 0)
    t = (m - 1.0) / (m + 1.0)
    t2 = t * t
    lnm = 2.0 * t * (1.0 + t2 * (1.0 / 3.0 + t2 * (0.2 + t2 * (1.0 / 7.0))))
    return e.astype(jnp.float32) * _LN2 + lnm


@functools.lru_cache(maxsize=None)
def _make_sc_gather(B: int):
    info = plsc.get_sparse_core_info()
    NC, NS, L = info.num_cores, info.num_subcores, info.num_lanes
    NW = NC * NS
    assert B % (8 * NW) == 0
    b_per_w = B // NW
    mesh = plsc.VectorSubcoreMesh(core_axis_name="c", subcore_axis_name="s")

    n_seg = b_per_w // 128

    @functools.partial(
        pl.kernel,
        out_type=jax.ShapeDtypeStruct((B,), jnp.float32),
        mesh=mesh,
        compiler_params=pltpu.CompilerParams(needs_layout_passes=False),
        scratch_types=[
            pltpu.VMEM((b_per_w, 2), jnp.int32),
            pltpu.VMEM((n_seg, 128), jnp.int32),
            pltpu.VMEM((n_seg, 128), jnp.float32),
            pltpu.VMEM((_V,), jnp.float32),
            pltpu.VMEM((b_per_w,), jnp.float32),
            pltpu.SemaphoreType.DMA,
        ],
    )
    def sc(inp_hbm, nd_hbm, probs_hbm, out_hbm, nd_v, idx_v, vals_v, logp_v,
           out_v, sem):
        wid = lax.axis_index("s") * NC + lax.axis_index("c")
        base = wid * b_per_w
        pltpu.sync_copy(nd_hbm.at[pl.ds(base, b_per_w), :], nd_v)
        pltpu.sync_copy(probs_hbm, logp_v)

        lanes = lax.iota(jnp.int32, L)
        col0 = jnp.zeros((L,), jnp.int32)
        col1 = jnp.ones((L,), jnp.int32)

        per_seg = 128 // L
        for g in range(n_seg):
            for u in range(per_seg):
                rows = (g * per_seg + u) * L + lanes
                r = plsc.load_gather(nd_v, [rows, col0])
                c = plsc.load_gather(nd_v, [rows, col1])
                idx_v[g, pl.ds(u * L, L)] = r * N_COLS + c

        copies = [
            pltpu.async_copy(inp_hbm.at[idx_v.at[g]], vals_v.at[g], sem)
            for g in range(n_seg)
        ]

        G = _V // L
        ps = [logp_v[pl.ds(g * L, L)] for g in range(G)]
        mv = ps[0]
        for p in ps[1:]:
            mv = jnp.maximum(mv, p)
        m = jnp.max(mv)
        sv = jnp.exp(ps[0] - m)
        for p in ps[1:]:
            sv = sv + jnp.exp(p - m)
        s_vec = jnp.broadcast_to(jnp.sum(sv), (L,))
        lse = m + _vln(s_vec)
        for g in range(G):
            logp_v[pl.ds(g * L, L)] = ps[g] - lse

        for c_ in copies:
            c_.wait()

        for g in range(n_seg):
            for u in range(per_seg):
                v = vals_v[g, pl.ds(u * L, L)]
                k = v.astype(jnp.int32)
                o = plsc.load_gather(logp_v, [k])
                out_v[pl.ds((g * per_seg + u) * L, L)] = o

        pltpu.sync_copy(out_v, out_hbm.at[pl.ds(base, b_per_w)])

    return sc


def kernel(inputs, nd_idxs, probs):
    B = inputs.shape[0]
    out = _make_sc_gather(B)(inputs.reshape(-1), nd_idxs, probs)
    return out.reshape(B, 1)


# speedup vs baseline: 1.3343x; 1.3343x over previous
import functools

import jax
import jax.numpy as jnp
from jax import lax
from jax.experimental import pallas as pl
from jax.experimental.pallas import tpu as pltpu
from jax.experimental.pallas import tpu_sc as plsc

_R = 200
_V = 128

_LN2 = 0.6931471805599453
_SQRT2 = 1.4142135623730951


def _vln(x):
    bits = plsc.bitcast(x, jnp.int32)
    e = (bits >> 23) - 127
    mbits = (bits & 0x007FFFFF) | 0x3F800000
    m = plsc.bitcast(mbits, jnp.float32)
    big = m > _SQRT2
    m = jnp.where(big, m * 0.5, m)
    e = e + jnp.where(big, 1, 0)
    t = (m - 1.0) / (m + 1.0)
    t2 = t * t
    lnm = 2.0 * t * (1.0 + t2 * (1.0 / 3.0 + t2 * (0.2 + t2 * (1.0 / 7.0))))
    return e.astype(jnp.float32) * _LN2 + lnm


@functools.lru_cache(maxsize=None)
def _make_sc_gather(B: int):
    info = plsc.get_sparse_core_info()
    NC, NS, L = info.num_cores, info.num_subcores, info.num_lanes
    NW = NC * NS
    assert B % (8 * NW) == 0
    b_per_w = B // NW
    groups = b_per_w // L
    mesh = plsc.VectorSubcoreMesh(core_axis_name="c", subcore_axis_name="s")

    @functools.partial(
        pl.kernel,
        out_type=jax.ShapeDtypeStruct((B,), jnp.float32),
        mesh=mesh,
        compiler_params=pltpu.CompilerParams(needs_layout_passes=False),
        scratch_types=[
            pltpu.VMEM((_R, _R), jnp.float32),
            pltpu.VMEM((b_per_w, 2), jnp.int32),
            pltpu.VMEM((_V,), jnp.float32),
            pltpu.VMEM((b_per_w,), jnp.float32),
            pltpu.SemaphoreType.DMA,
        ],
    )
    def sc(inp_hbm, nd_hbm, probs_hbm, out_hbm, tab_v, nd_v, logp_v, out_v,
           sem):
        wid = lax.axis_index("s") * NC + lax.axis_index("c")
        base = wid * b_per_w
        tab_cp = pltpu.async_copy(inp_hbm.at[pl.ds(0, _R), :], tab_v, sem)
        nd_cp = pltpu.async_copy(nd_hbm.at[pl.ds(base, b_per_w), :], nd_v, sem)
        pltpu.sync_copy(probs_hbm, logp_v)

        G = _V // L
        ps = [logp_v[pl.ds(g * L, L)] for g in range(G)]
        mv = ps[0]
        for p in ps[1:]:
            mv = jnp.maximum(mv, p)
        m = jnp.max(mv)
        sv = jnp.exp(ps[0] - m)
        for p in ps[1:]:
            sv = sv + jnp.exp(p - m)
        s_vec = jnp.broadcast_to(jnp.sum(sv), (L,))
        lse = m + _vln(s_vec)
        for g in range(G):
            logp_v[pl.ds(g * L, L)] = ps[g] - lse

        nd_cp.wait()
        tab_cp.wait()

        lanes = lax.iota(jnp.int32, L)
        col0 = jnp.zeros((L,), jnp.int32)
        col1 = jnp.ones((L,), jnp.int32)

        for j in range(groups):
            rows = j * L + lanes
            r = plsc.load_gather(nd_v, [rows, col0])
            c = plsc.load_gather(nd_v, [rows, col1])
            v = plsc.load_gather(tab_v, [r, c])
            k = v.astype(jnp.int32)
            o = plsc.load_gather(logp_v, [k])
            out_v[pl.ds(j * L, L)] = o

        pltpu.sync_copy(out_v, out_hbm.at[pl.ds(base, b_per_w)])

    return sc


def kernel(inputs, nd_idxs, probs):
    B = inputs.shape[0]
    out = _make_sc_gather(B)(inputs, nd_idxs, probs)
    return out.reshape(B, 1)


# speedup vs baseline: 1.8453x; 1.3830x over previous
import functools

import jax
import jax.numpy as jnp
from jax import lax
from jax.experimental import pallas as pl
from jax.experimental.pallas import tpu as pltpu
from jax.experimental.pallas import tpu_sc as plsc

_R = 200
_V = 128

_LN2 = 0.6931471805599453
_SQRT2 = 1.4142135623730951


def _vln(x):
    bits = plsc.bitcast(x, jnp.int32)
    e = (bits >> 23) - 127
    mbits = (bits & 0x007FFFFF) | 0x3F800000
    m = plsc.bitcast(mbits, jnp.float32)
    big = m > _SQRT2
    m = jnp.where(big, m * 0.5, m)
    e = e + jnp.where(big, 1, 0)
    t = (m - 1.0) / (m + 1.0)
    t2 = t * t
    lnm = 2.0 * t * (1.0 + t2 * (1.0 / 3.0 + t2 * (0.2 + t2 * (1.0 / 7.0))))
    return e.astype(jnp.float32) * _LN2 + lnm


@functools.lru_cache(maxsize=None)
def _make_sc_gather(B: int):
    info = plsc.get_sparse_core_info()
    NC, NS, L = info.num_cores, info.num_subcores, info.num_lanes
    NW = NC * NS
    assert B % (8 * NW) == 0
    b_per_w = B // NW
    groups = b_per_w // L
    mesh = plsc.VectorSubcoreMesh(core_axis_name="c", subcore_axis_name="s")

    @functools.partial(
        pl.kernel,
        out_type=jax.ShapeDtypeStruct((B,), jnp.float32),
        mesh=mesh,
        compiler_params=pltpu.CompilerParams(needs_layout_passes=False),
        scratch_types=[
            pltpu.VMEM((_R, _R), jnp.float32),
            pltpu.VMEM((b_per_w, 2), jnp.int32),
            pltpu.VMEM((_V,), jnp.float32),
            pltpu.VMEM((b_per_w,), jnp.float32),
            pltpu.SemaphoreType.DMA,
        ],
    )
    def sc(inp_hbm, nd_hbm, probs_hbm, out_hbm, tab_v, nd_v, logp_v, out_v,
           sem):
        wid = lax.axis_index("s") * NC + lax.axis_index("c")
        base = wid * b_per_w
        tab_cp = pltpu.async_copy(inp_hbm, tab_v, sem)
        nd_cp = pltpu.async_copy(nd_hbm.at[pl.ds(base, b_per_w), :], nd_v, sem)
        pltpu.sync_copy(probs_hbm, logp_v)

        G = _V // L
        ps = [logp_v[pl.ds(g * L, L)] for g in range(G)]
        mv = ps[0]
        for p in ps[1:]:
            mv = jnp.maximum(mv, p)
        m = jnp.max(mv)
        sv = jnp.exp(ps[0] - m)
        for p in ps[1:]:
            sv = sv + jnp.exp(p - m)
        s_vec = jnp.broadcast_to(jnp.sum(sv), (L,))
        lse = m + _vln(s_vec)
        for g in range(G):
            logp_v[pl.ds(g * L, L)] = ps[g] - lse

        nd_cp.wait()
        tab_cp.wait()

        lanes = lax.iota(jnp.int32, L)
        col0 = jnp.zeros((L,), jnp.int32)
        col1 = jnp.ones((L,), jnp.int32)

        for j in range(groups):
            rows = j * L + lanes
            r = plsc.load_gather(nd_v, [rows, col0])
            c = plsc.load_gather(nd_v, [rows, col1])
            v = plsc.load_gather(tab_v, [r, c])
            k = v.astype(jnp.int32)
            o = plsc.load_gather(logp_v, [k])
            out_v[pl.ds(j * L, L)] = o

        pltpu.sync_copy(out_v, out_hbm.at[pl.ds(base, b_per_w)])

    return sc


def kernel(inputs, nd_idxs, probs):
    B = inputs.shape[0]
    out = _make_sc_gather(B)(inputs[:_R, :_R], nd_idxs, probs)
    return out.reshape(B, 1)


# speedup vs baseline: 2.3319x; 1.2637x over previous
import functools

import jax
import jax.numpy as jnp
from jax import lax
from jax.experimental import pallas as pl
from jax.experimental.pallas import tpu as pltpu
from jax.experimental.pallas import tpu_sc as plsc

_R = 200
_V = 128

_LN2 = 0.6931471805599453
_SQRT2 = 1.4142135623730951


def _vln(x):
    bits = plsc.bitcast(x, jnp.int32)
    e = (bits >> 23) - 127
    mbits = (bits & 0x007FFFFF) | 0x3F800000
    m = plsc.bitcast(mbits, jnp.float32)
    big = m > _SQRT2
    m = jnp.where(big, m * 0.5, m)
    e = e + jnp.where(big, 1, 0)
    t = (m - 1.0) / (m + 1.0)
    t2 = t * t
    lnm = 2.0 * t * (1.0 + t2 * (1.0 / 3.0 + t2 * (0.2 + t2 * (1.0 / 7.0))))
    return e.astype(jnp.float32) * _LN2 + lnm


@functools.lru_cache(maxsize=None)
def _make_sc_gather(B: int):
    info = plsc.get_sparse_core_info()
    NC, NS, L = info.num_cores, info.num_subcores, info.num_lanes
    NW = NC * NS
    assert B % (8 * NW) == 0
    b_per_w = B // NW
    groups = b_per_w // L
    mesh = plsc.VectorSubcoreMesh(core_axis_name="c", subcore_axis_name="s")

    @functools.partial(
        pl.kernel,
        out_type=jax.ShapeDtypeStruct((B,), jnp.float32),
        mesh=mesh,
        compiler_params=pltpu.CompilerParams(needs_layout_passes=False),
        scratch_types=[
            pltpu.VMEM((_R, _R), jnp.float32),
            pltpu.VMEM((b_per_w,), jnp.int32),
            pltpu.VMEM((b_per_w,), jnp.int32),
            pltpu.VMEM((_V,), jnp.float32),
            pltpu.VMEM((b_per_w,), jnp.float32),
            pltpu.SemaphoreType.DMA,
        ],
    )
    def sc(inp_hbm, nd0_hbm, nd1_hbm, probs_hbm, out_hbm, tab_v, nd0_v, nd1_v,
           logp_v, out_v, sem):
        wid = lax.axis_index("s") * NC + lax.axis_index("c")
        base = wid * b_per_w
        tab_cp = pltpu.async_copy(inp_hbm, tab_v, sem)
        nd0_cp = pltpu.async_copy(nd0_hbm.at[pl.ds(base, b_per_w)], nd0_v, sem)
        nd1_cp = pltpu.async_copy(nd1_hbm.at[pl.ds(base, b_per_w)], nd1_v, sem)
        pltpu.sync_copy(probs_hbm, logp_v)

        G = _V // L
        ps = [logp_v[pl.ds(g * L, L)] for g in range(G)]
        mv = ps[0]
        for p in ps[1:]:
            mv = jnp.maximum(mv, p)
        m = jnp.max(mv)
        sv = jnp.exp(ps[0] - m)
        for p in ps[1:]:
            sv = sv + jnp.exp(p - m)
        s_vec = jnp.broadcast_to(jnp.sum(sv), (L,))
        lse = m + _vln(s_vec)
        for g in range(G):
            logp_v[pl.ds(g * L, L)] = ps[g] - lse

        nd0_cp.wait()
        nd1_cp.wait()
        tab_cp.wait()

        for j in range(groups):
            r = nd0_v[pl.ds(j * L, L)]
            c = nd1_v[pl.ds(j * L, L)]
            v = plsc.load_gather(tab_v, [r, c])
            k = v.astype(jnp.int32)
            o = plsc.load_gather(logp_v, [k])
            out_v[pl.ds(j * L, L)] = o

        pltpu.sync_copy(out_v, out_hbm.at[pl.ds(base, b_per_w)])

    return sc


def kernel(inputs, nd_idxs, probs):
    B = inputs.shape[0]
    out = _make_sc_gather(B)(
        inputs[:_R, :_R], nd_idxs[:, 0], nd_idxs[:, 1], probs)
    return out.reshape(B, 1)


# speedup vs baseline: 2.6072x; 1.1181x over previous
import functools

import jax
import jax.numpy as jnp
from jax import lax
from jax.experimental import pallas as pl
from jax.experimental.pallas import tpu as pltpu
from jax.experimental.pallas import tpu_sc as plsc

_R = 200
_V = 128

_LN2 = 0.6931471805599453
_SQRT2 = 1.4142135623730951


def _vln(x):
    bits = plsc.bitcast(x, jnp.int32)
    e = (bits >> 23) - 127
    mbits = (bits & 0x007FFFFF) | 0x3F800000
    m = plsc.bitcast(mbits, jnp.float32)
    big = m > _SQRT2
    m = jnp.where(big, m * 0.5, m)
    e = e + jnp.where(big, 1, 0)
    t = (m - 1.0) / (m + 1.0)
    t2 = t * t
    lnm = 2.0 * t * (1.0 + t2 * (1.0 / 3.0 + t2 * (0.2 + t2 * (1.0 / 7.0))))
    return e.astype(jnp.float32) * _LN2 + lnm


@functools.lru_cache(maxsize=None)
def _make_sc_gather(B: int):
    info = plsc.get_sparse_core_info()
    NC, NS, L = info.num_cores, info.num_subcores, info.num_lanes
    NW = NC * NS
    assert B % (8 * NW) == 0
    b_per_w = B // NW
    groups = b_per_w // L
    mesh = plsc.VectorSubcoreMesh(core_axis_name="c", subcore_axis_name="s")

    @functools.partial(
        pl.kernel,
        out_type=jax.ShapeDtypeStruct((B,), jnp.float32),
        mesh=mesh,
        compiler_params=pltpu.CompilerParams(needs_layout_passes=False),
        scratch_types=[
            pltpu.VMEM((_R, _R), jnp.float32),
            pltpu.VMEM_SHARED((_R, _R), jnp.float32),
            pltpu.VMEM((b_per_w,), jnp.int32),
            pltpu.VMEM((b_per_w,), jnp.int32),
            pltpu.VMEM((_V,), jnp.float32),
            pltpu.VMEM((b_per_w,), jnp.float32),
            pltpu.SemaphoreType.DMA,
        ],
    )
    def sc(inp_hbm, nd0_hbm, nd1_hbm, probs_hbm, out_hbm, tab_v, tab_sh,
           nd0_v, nd1_v, logp_v, out_v, sem):
        sid = lax.axis_index("s")
        wid = sid * NC + lax.axis_index("c")
        base = wid * b_per_w
        @pl.when(sid == 0)
        def _():
            pltpu.sync_copy(inp_hbm, tab_sh)

        nd0_cp = pltpu.async_copy(nd0_hbm.at[pl.ds(base, b_per_w)], nd0_v, sem)
        nd1_cp = pltpu.async_copy(nd1_hbm.at[pl.ds(base, b_per_w)], nd1_v, sem)
        pltpu.sync_copy(probs_hbm, logp_v)

        G = _V // L
        ps = [logp_v[pl.ds(g * L, L)] for g in range(G)]
        mv = ps[0]
        for p in ps[1:]:
            mv = jnp.maximum(mv, p)
        m = jnp.max(mv)
        sv = jnp.exp(ps[0] - m)
        for p in ps[1:]:
            sv = sv + jnp.exp(p - m)
        s_vec = jnp.broadcast_to(jnp.sum(sv), (L,))
        lse = m + _vln(s_vec)
        for g in range(G):
            logp_v[pl.ds(g * L, L)] = ps[g] - lse

        plsc.subcore_barrier()
        pltpu.sync_copy(tab_sh, tab_v)
        nd0_cp.wait()
        nd1_cp.wait()

        for j in range(groups):
            r = nd0_v[pl.ds(j * L, L)]
            c = nd1_v[pl.ds(j * L, L)]
            v = plsc.load_gather(tab_v, [r, c])
            k = v.astype(jnp.int32)
            o = plsc.load_gather(logp_v, [k])
            out_v[pl.ds(j * L, L)] = o

        pltpu.sync_copy(out_v, out_hbm.at[pl.ds(base, b_per_w)])

    return sc


def kernel(inputs, nd_idxs, probs):
    B = inputs.shape[0]
    out = _make_sc_gather(B)(
        inputs[:_R, :_R], nd_idxs[:, 0], nd_idxs[:, 1], probs)
    return out.reshape(B, 1)
